# baseline (speedup 1.0000x reference)
import functools

import jax
import jax.numpy as jnp
from jax import lax
from jax.experimental import pallas as pl
from jax.experimental.pallas import tpu as pltpu
from jax.experimental.pallas import tpu_sc as plsc

N = 10000
NP = 10240
D = 128
E = 320000
HD = D // 2
NC = 2
NS = 16
R = NP // NS
CHUNK = 128
GC = 16
GROUPS = 10
NBUF = 3
NCHUNK = GC * GROUPS
EP = NS * NCHUNK * CHUNK


def _body(emb2, edata, zeros_h, out, cur,
          spN, rows0, rows1, rows2, edv,
          sg0, sg1, sg2, ss0, ss1, ss2):
    cid = lax.axis_index("c")
    sid = lax.axis_index("s")
    row0 = sid * R
    rslice = pl.ds(row0, R)
    rows = [rows0, rows1, rows2]
    sg = [sg0, sg1, sg2]
    ss = [ss0, ss1, ss2]

    pltpu.sync_copy(zeros_h.at[rslice], spN.at[rslice])
    plsc.subcore_barrier()

    def scale(buf, g2):
        def scale_body(g, _):
            wv = plsc.bitcast(edv[g2, 2, pl.ds(g * 16, 16)], jnp.float32)
            for t in range(16):
                w = wv[t]
                i = g * 16 + t
                for k in range(4):
                    sl = pl.ds(k * 16, 16)
                    buf[i, sl] = buf[i, sl] * w
            return 0

        lax.fori_loop(0, CHUNK // 16, scale_body, 0)

    def do_layer(gsrc):
        def group_body(gj, _):
            pltpu.sync_copy(edata.at[sid, pl.ds(gj * GC, GC)], edv)
            gd = [None] * GC
            sd = [None] * GC
            waited = [False] * GC
            for c in range(GC + 2):
                f = c - NBUF
                if f >= 0 and sd[f] is not None and not waited[f]:
                    sd[f].wait()
                    waited[f] = True
                if c < GC:
                    b = c % NBUF
                    gd[c] = pltpu.async_copy(
                        gsrc.at[edv.at[c, 0]], rows[b], sg[b])
                if c >= 2:
                    p = c - 2
                    b = p % NBUF
                    gd[p].wait()
                    scale(rows[b], p)
                    sd[p] = pltpu.async_copy(
                        rows[b], spN.at[edv.at[p, 1]], ss[b], add=True)
            for p in range(GC):
                if sd[p] is not None and not waited[p]:
                    sd[p].wait()
            return 0

        lax.fori_loop(0, GROUPS, group_body, 0)

    def finish_layer(first, last):
        for c5 in range(R // CHUNK):
            sl_r = pl.ds(row0 + c5 * CHUNK, CHUNK)
            pltpu.sync_copy(spN.at[sl_r], rows0)
            if first:
                pltpu.sync_copy(emb2.at[cid, sl_r], rows1)
            else:
                pltpu.sync_copy(out.at[cid, sl_r], rows1)

            def add_body(i, _):
                for k in range(4):
                    sl = pl.ds(k * 16, 16)
                    v = rows1[i, sl] + rows0[i, sl]
                    if last:
                        v = v * 0.25
                    rows1[i, sl] = v
                return 0

            lax.fori_loop(0, CHUNK, add_body, 0)
            pltpu.sync_copy(rows1, out.at[cid, sl_r])
            if not last:
                pltpu.sync_copy(rows0, cur.at[cid, sl_r])
        if not last:
            pltpu.sync_copy(zeros_h.at[rslice], spN.at[rslice])

    do_layer(emb2.at[cid])
    plsc.subcore_barrier()
    finish_layer(first=True, last=False)
    plsc.subcore_barrier()

    do_layer(cur.at[cid])
    plsc.subcore_barrier()
    finish_layer(first=False, last=False)
    plsc.subcore_barrier()

    do_layer(cur.at[cid])
    plsc.subcore_barrier()
    finish_layer(first=False, last=True)


_sc_kernel = functools.partial(
    pl.kernel,
    out_type=(jax.ShapeDtypeStruct((NC, NP, HD), jnp.float32),
              jax.ShapeDtypeStruct((NC, NP, HD), jnp.float32)),
    mesh=plsc.VectorSubcoreMesh(core_axis_name="c", subcore_axis_name="s"),
    compiler_params=pltpu.CompilerParams(use_tc_tiling_on_sc=False,
                                         needs_layout_passes=False),
    scratch_types=[
        pltpu.VMEM_SHARED((NP, HD), jnp.float32),
        pltpu.VMEM((CHUNK, HD), jnp.float32),
        pltpu.VMEM((CHUNK, HD), jnp.float32),
        pltpu.VMEM((CHUNK, HD), jnp.float32),
        pltpu.VMEM((GC, 3, CHUNK), jnp.int32),
        pltpu.SemaphoreType.DMA,
        pltpu.SemaphoreType.DMA,
        pltpu.SemaphoreType.DMA,
        pltpu.SemaphoreType.DMA,
        pltpu.SemaphoreType.DMA,
        pltpu.SemaphoreType.DMA,
    ],
)(_body)


@jax.jit
def kernel(all_emb, edge_index, edge_weight):
    src = edge_index[0]
    dst = edge_index[1]
    pad = EP - E
    src_p = jnp.pad(src, (0, pad)).reshape(NS, NCHUNK, CHUNK)
    dst_p = jnp.pad(dst, (0, pad)).reshape(NS, NCHUNK, CHUNK)
    w_p = jnp.pad(edge_weight, (0, pad)).reshape(NS, NCHUNK, CHUNK)
    edata = jnp.stack(
        [src_p, dst_p, jax.lax.bitcast_convert_type(w_p, jnp.int32)], axis=2)
    emb_p = jnp.pad(all_emb, ((0, NP - N), (0, 0)))
    emb2 = jnp.stack([emb_p[:, :HD], emb_p[:, HD:]])
    zeros_h = jnp.zeros((NP, HD), jnp.float32)
    out, _ = _sc_kernel(emb2, edata, zeros_h)
    return out[:, :N, :].transpose(1, 0, 2).reshape(N, D)
